# baseline (speedup 1.0000x reference)
import functools

import jax
import jax.numpy as jnp
from jax import lax
from jax.experimental import pallas as pl
from jax.experimental.pallas import tpu as pltpu
from jax.experimental.pallas import tpu_sc as plsc

N_NODES = 100000
N_CLUSTERS = 10000
NNZ = 400000
D_FEAT = 128
DG = D_FEAT // 16

N_SEC = 12
SEC_ROWS = 8960
CHUNK = 128
N_CHUNKS = 3200
PADN = N_CHUNKS * CHUNK
CAP_CH = 32
CAP_E = CAP_CH * CHUNK
PAD_ARR = PADN + CAP_E
FLUSH_ROWS = 80
N_FLUSH = SEC_ROWS // FLUSH_ROWS
ZERO_ROWS = 40
N_ZERO = SEC_ROWS // ZERO_ROWS


def _body(xp, valp, rowp, colp, kb, out, acc, g0, g1, rowv, colv, valv,
          kbv, lrow0, lrow1, zerov, semi, semg0, semg1, semsc0, semsc1):
    c = lax.axis_index("c")
    t = lax.axis_index("s")
    zeros16 = jnp.zeros((16,), jnp.float32)
    gbufs = (g0, g1)
    lrows = (lrow0, lrow1)
    semgs = (semg0, semg1)
    semscs = (semsc0, semsc1)

    def zrow(r, carry):
        for d in range(DG):
            zerov[r, pl.ds(d * 16, 16)] = zeros16
        return carry

    lax.fori_loop(0, ZERO_ROWS, zrow, 0)

    pltpu.sync_copy(kb, kbv)
    kb16 = kbv[pl.ds(c * 16, 16)]

    for i in range(N_SEC // 2):
        s = c + 2 * i
        sec_lo = s * SEC_ROWS
        sec_hi = sec_lo + SEC_ROWS

        zdescs = [
            pltpu.async_copy(
                zerov, acc.at[pl.ds((j * 16 + t) * ZERO_ROWS, ZERO_ROWS)],
                semi) for j in range(N_ZERO // 16)
        ]

        k_lo = kb16[2 * i]
        k_hi = kb16[2 * i + 1]
        per_tile = (k_hi - k_lo + 15) // 16
        my_lo = k_lo + t * per_tile
        my_hi = jnp.minimum(my_lo + per_tile, k_hi)
        nch_all = jnp.maximum(my_hi - my_lo, 0)

        for d in zdescs:
            d.wait()
        plsc.subcore_barrier()

        def pass_body(p, carry2):
            kb = my_lo + p * CAP_CH
            m = jnp.minimum(my_hi - kb, CAP_CH)
            eb = kb * CHUNK
            d1 = pltpu.async_copy(rowp.at[pl.ds(eb, CAP_E)], rowv, semi)
            d2 = pltpu.async_copy(colp.at[pl.ds(eb, CAP_E)], colv, semi)
            d3 = pltpu.async_copy(valp.at[pl.ds(eb, CAP_E)], valv, semi)
            d1.wait()
            d2.wait()
            d3.wait()

            def gather(j, parity):
                jc = jnp.minimum(j, m - 1)
                pltpu.async_copy(xp.at[colv.at[pl.ds(jc * CHUNK, CHUNK)]],
                                 gbufs[parity], semgs[parity])

            gather(jnp.int32(0), 0)

            def pair_body(q, carry3):
                for b in range(2):
                    j = 2 * q + b

                    @pl.when((j >= 1) & (j - 1 < m))
                    def _():
                        pltpu.make_async_copy(gbufs[b ^ 1],
                                              acc.at[lrows[b ^ 1]],
                                              semscs[b ^ 1]).wait()

                    gather(j + 1, b ^ 1)
                    pltpu.make_async_copy(xp.at[pl.ds(0, CHUNK)], gbufs[b],
                                          semgs[b]).wait()

                    def grp(g, carry4):
                        base = j * CHUNK + g * 16
                        r16 = rowv[pl.ds(base, 16)]
                        ok = (r16 >= sec_lo) & (r16 < sec_hi)
                        lrows[b][pl.ds(g * 16, 16)] = jnp.where(
                            ok, r16 - sec_lo, 0)
                        v16 = jnp.where(ok, valv[pl.ds(base, 16)], 0.0)
                        for lane in range(16):
                            v = v16[lane]
                            r = g * 16 + lane
                            for d in range(DG):
                                gbufs[b][r, pl.ds(d * 16, 16)] = (
                                    gbufs[b][r, pl.ds(d * 16, 16)] * v)
                        return carry4

                    lax.fori_loop(0, CHUNK // 16, grp, 0)

                    @pl.when(j < m)
                    def _():
                        pltpu.async_copy(gbufs[b], acc.at[lrows[b]],
                                         semscs[b], add=True)
                return carry3

            lax.fori_loop(0, (m + 1) // 2, pair_body, 0)
            pltpu.make_async_copy(xp.at[pl.ds(0, CHUNK)], gbufs[0],
                                  semgs[0]).wait()

            @pl.when(m % 2 == 0)
            def _():
                pltpu.make_async_copy(gbufs[1], acc.at[lrows[1]],
                                      semscs[1]).wait()

            return carry2

        lax.fori_loop(0, (nch_all + CAP_CH - 1) // CAP_CH, pass_body, 0)
        plsc.subcore_barrier()

        if i < N_SEC // 2 - 1:
            fdescs = []
            for j in range(N_FLUSH // 16):
                b = j * 16 + t
                fdescs.append(
                    pltpu.async_copy(
                        acc.at[pl.ds(b * FLUSH_ROWS, FLUSH_ROWS)],
                        out.at[pl.ds(sec_lo + b * FLUSH_ROWS, FLUSH_ROWS)],
                        semi))
            for d in fdescs:
                d.wait()
        else:
            nvalid = (N_NODES - sec_lo + FLUSH_ROWS - 1) // FLUSH_ROWS
            for j in range(N_FLUSH // 16):
                b = j * 16 + t

                @pl.when(b < nvalid)
                def _():
                    pltpu.sync_copy(
                        acc.at[pl.ds(b * FLUSH_ROWS, FLUSH_ROWS)],
                        out.at[pl.ds(sec_lo + b * FLUSH_ROWS, FLUSH_ROWS)])

        plsc.subcore_barrier()


@functools.partial(pl.kernel,
                   out_type=jax.ShapeDtypeStruct((N_NODES, D_FEAT),
                                                 jnp.float32),
                   mesh=plsc.VectorSubcoreMesh(core_axis_name="c",
                                               subcore_axis_name="s"),
                   scratch_types=[
                       pltpu.VMEM_SHARED((SEC_ROWS, D_FEAT), jnp.float32),
                       pltpu.VMEM((CHUNK, D_FEAT), jnp.float32),
                       pltpu.VMEM((CHUNK, D_FEAT), jnp.float32),
                       pltpu.VMEM((CAP_E,), jnp.int32),
                       pltpu.VMEM((CAP_E,), jnp.int32),
                       pltpu.VMEM((CAP_E,), jnp.float32),
                       pltpu.VMEM((32,), jnp.int32),
                       pltpu.VMEM((CHUNK,), jnp.int32),
                       pltpu.VMEM((CHUNK,), jnp.int32),
                       pltpu.VMEM((ZERO_ROWS, D_FEAT), jnp.float32),
                       pltpu.SemaphoreType.DMA,
                       pltpu.SemaphoreType.DMA,
                       pltpu.SemaphoreType.DMA,
                       pltpu.SemaphoreType.DMA,
                       pltpu.SemaphoreType.DMA,
                   ])
def _lift(xp, valp, rowp, colp, kb, out, acc, g0, g1, rowv, colv, valv,
          kbv, lrow0, lrow1, zerov, semi, semg0, semg1, semsc0, semsc1):
    _body(xp, valp, rowp, colp, kb, out, acc, g0, g1, rowv, colv, valv,
          kbv, lrow0, lrow1, zerov, semi, semg0, semg1, semsc0, semsc1)


def kernel(x_pool, values, row, col):
    row = row.astype(jnp.int32)
    col = col.astype(jnp.int32)
    pad = PAD_ARR - NNZ
    rowp = jnp.concatenate([row, jnp.full((pad,), N_NODES, jnp.int32)])
    colp = jnp.concatenate([col, jnp.zeros((pad,), jnp.int32)])
    valp = jnp.concatenate([values, jnp.zeros((pad,), jnp.float32)])
    cf = rowp[:PADN:CHUNK]
    cl = rowp[CHUNK - 1:PADN:CHUNK]
    sec_lo = jnp.arange(N_SEC, dtype=jnp.int32) * SEC_ROWS
    k_lo = jnp.searchsorted(cl, sec_lo, side="left").astype(jnp.int32)
    k_hi = jnp.searchsorted(cf, sec_lo + SEC_ROWS,
                            side="left").astype(jnp.int32)
    k_hi = jnp.minimum(k_hi, (NNZ + CHUNK - 1) // CHUNK)
    s_of = jnp.array([[0, 2, 4, 6, 8, 10], [1, 3, 5, 7, 9, 11]],
                     dtype=jnp.int32)
    kb = jnp.stack([k_lo[s_of], k_hi[s_of]], axis=-1).reshape(2, 12)
    kb = jnp.pad(kb, ((0, 0), (0, 4))).reshape(32)
    return _lift(x_pool, valp, rowp, colp, kb)


# speedup vs baseline: 1.0579x; 1.0579x over previous
import functools

import jax
import jax.numpy as jnp
from jax import lax
from jax.experimental import pallas as pl
from jax.experimental.pallas import tpu as pltpu
from jax.experimental.pallas import tpu_sc as plsc

N_NODES = 100000
N_CLUSTERS = 10000
NNZ = 400000
D_FEAT = 128
DG = D_FEAT // 16

N_SEC = 14
SEC_ROWS = 7360
CHUNK = 128
N_CHUNKS = 3200
PADN = N_CHUNKS * CHUNK
CAP_CH = 32
CAP_E = CAP_CH * CHUNK
PAD_ARR = PADN + CAP_E
FLUSH_ROWS = 80
N_FLUSH = SEC_ROWS // FLUSH_ROWS
ZERO_ROWS = 40
N_ZERO = SEC_ROWS // ZERO_ROWS


def _body(xp, valp, rowp, colp, kb, out, acc, g0, g1, g2, rowv, colv, valv,
          kbv, lrow0, lrow1, lrow2, zerov, semi, semp, semg0, semg1,
          semg2, semsc0, semsc1, semsc2):
    c = lax.axis_index("c")
    t = lax.axis_index("s")
    zeros16 = jnp.zeros((16,), jnp.float32)
    gbufs = (g0, g1, g2)
    lrows = (lrow0, lrow1, lrow2)
    semgs = (semg0, semg1, semg2)
    semscs = (semsc0, semsc1, semsc2)

    def zrow(r, carry):
        for d in range(DG):
            zerov[r, pl.ds(d * 16, 16)] = zeros16
        return carry

    lax.fori_loop(0, ZERO_ROWS, zrow, 0)

    pltpu.sync_copy(kb, kbv)
    kb16 = kbv[pl.ds(c * 16, 16)]

    for i in range(N_SEC // 2):
        s = c + 2 * i
        sec_lo = s * SEC_ROWS
        sec_hi = sec_lo + SEC_ROWS

        zdescs = [
            pltpu.async_copy(
                zerov, acc.at[pl.ds((j * 16 + t) * ZERO_ROWS, ZERO_ROWS)],
                semi) for j in range(N_ZERO // 16)
        ]

        @pl.when((N_ZERO // 16) * 16 + t < N_ZERO)
        def _():
            pltpu.sync_copy(
                zerov,
                acc.at[pl.ds(((N_ZERO // 16) * 16 + t) * ZERO_ROWS,
                             ZERO_ROWS)])

        k_lo = kb16[2 * i]
        k_hi = kb16[2 * i + 1]
        per_tile = (k_hi - k_lo + 15) // 16
        my_lo = k_lo + t * per_tile
        my_hi = jnp.minimum(my_lo + per_tile, k_hi)
        nch_all = jnp.maximum(my_hi - my_lo, 0)

        def preload(kb2):
            ebp = jnp.minimum(kb2, N_CHUNKS) * CHUNK
            pltpu.async_copy(rowp.at[pl.ds(ebp, CAP_E)], rowv, semp)
            pltpu.async_copy(colp.at[pl.ds(ebp, CAP_E)], colv, semp)
            pltpu.async_copy(valp.at[pl.ds(ebp, CAP_E)], valv, semp)

        preload(my_lo)

        for d in zdescs:
            d.wait()
        plsc.subcore_barrier()

        def pass_body(p, carry2):
            kb = my_lo + p * CAP_CH
            m = jnp.minimum(my_hi - kb, CAP_CH)
            pltpu.make_async_copy(rowp.at[pl.ds(0, CAP_E)], rowv,
                                  semp).wait()
            pltpu.make_async_copy(colp.at[pl.ds(0, CAP_E)], colv,
                                  semp).wait()
            pltpu.make_async_copy(valp.at[pl.ds(0, CAP_E)], valv,
                                  semp).wait()

            def gather(j, parity):
                jc = jnp.minimum(j, m - 1)
                pltpu.async_copy(xp.at[colv.at[pl.ds(jc * CHUNK, CHUNK)]],
                                 gbufs[parity], semgs[parity])

            gather(jnp.int32(0), 0)

            def tri_body(q, carry3):
                for b in range(3):
                    j = 3 * q + b
                    bn = (b + 1) % 3

                    @pl.when((j >= 2) & (j - 2 < m))
                    def _():
                        pltpu.make_async_copy(gbufs[bn], acc.at[lrows[bn]],
                                              semscs[bn]).wait()

                    gather(j + 1, bn)
                    pltpu.make_async_copy(xp.at[pl.ds(0, CHUNK)], gbufs[b],
                                          semgs[b]).wait()

                    jr = jnp.minimum(j, m - 1)

                    def grp(g, carry4):
                        base = jr * CHUNK + g * 16
                        r16 = rowv[pl.ds(base, 16)]
                        ok = (r16 >= sec_lo) & (r16 < sec_hi)
                        lrows[b][pl.ds(g * 16, 16)] = jnp.where(
                            ok, r16 - sec_lo, 0)
                        v16 = jnp.where(ok, valv[pl.ds(base, 16)], 0.0)
                        for lane in range(16):
                            v = v16[lane]
                            r = g * 16 + lane
                            for d in range(DG):
                                gbufs[b][r, pl.ds(d * 16, 16)] = (
                                    gbufs[b][r, pl.ds(d * 16, 16)] * v)
                        return carry4

                    lax.fori_loop(0, CHUNK // 16, grp, 0)

                    @pl.when(j < m)
                    def _():
                        pltpu.async_copy(gbufs[b], acc.at[lrows[b]],
                                         semscs[b], add=True)
                return carry3

            lax.fori_loop(0, (m + 2) // 3, tri_body, 0)
            pltpu.make_async_copy(xp.at[pl.ds(0, CHUNK)], gbufs[0],
                                  semgs[0]).wait()

            @pl.when(m % 3 == 0)
            def _():
                pltpu.make_async_copy(gbufs[1], acc.at[lrows[1]],
                                      semscs[1]).wait()
                pltpu.make_async_copy(gbufs[2], acc.at[lrows[2]],
                                      semscs[2]).wait()

            @pl.when(m % 3 == 2)
            def _():
                pltpu.make_async_copy(gbufs[1], acc.at[lrows[1]],
                                      semscs[1]).wait()

            preload(my_lo + (p + 1) * CAP_CH)
            return carry2

        lax.fori_loop(0, (nch_all + CAP_CH - 1) // CAP_CH, pass_body, 0)
        pltpu.make_async_copy(rowp.at[pl.ds(0, CAP_E)], rowv, semp).wait()
        pltpu.make_async_copy(colp.at[pl.ds(0, CAP_E)], colv, semp).wait()
        pltpu.make_async_copy(valp.at[pl.ds(0, CAP_E)], valv, semp).wait()
        plsc.subcore_barrier()

        nvalid = (jnp.minimum(sec_hi, N_NODES) - sec_lo +
                  FLUSH_ROWS - 1) // FLUSH_ROWS
        if i < N_SEC // 2 - 1:
            fdescs = []
            for j in range(N_FLUSH // 16):
                b = j * 16 + t
                fdescs.append(
                    pltpu.async_copy(
                        acc.at[pl.ds(b * FLUSH_ROWS, FLUSH_ROWS)],
                        out.at[pl.ds(sec_lo + b * FLUSH_ROWS, FLUSH_ROWS)],
                        semi))
            for d in fdescs:
                d.wait()

            @pl.when((N_FLUSH // 16) * 16 + t < nvalid)
            def _():
                b2 = (N_FLUSH // 16) * 16 + t
                pltpu.sync_copy(
                    acc.at[pl.ds(b2 * FLUSH_ROWS, FLUSH_ROWS)],
                    out.at[pl.ds(sec_lo + b2 * FLUSH_ROWS, FLUSH_ROWS)])
        else:
            for j in range(N_FLUSH // 16 + 1):
                b = j * 16 + t

                @pl.when(b < nvalid)
                def _():
                    pltpu.sync_copy(
                        acc.at[pl.ds(b * FLUSH_ROWS, FLUSH_ROWS)],
                        out.at[pl.ds(sec_lo + b * FLUSH_ROWS, FLUSH_ROWS)])

        plsc.subcore_barrier()


@functools.partial(pl.kernel,
                   out_type=jax.ShapeDtypeStruct((N_NODES, D_FEAT),
                                                 jnp.float32),
                   mesh=plsc.VectorSubcoreMesh(core_axis_name="c",
                                               subcore_axis_name="s"),
                   scratch_types=[
                       pltpu.VMEM_SHARED((SEC_ROWS, D_FEAT), jnp.float32),
                       pltpu.VMEM((CHUNK, D_FEAT), jnp.float32),
                       pltpu.VMEM((CHUNK, D_FEAT), jnp.float32),
                       pltpu.VMEM((CHUNK, D_FEAT), jnp.float32),
                       pltpu.VMEM((CAP_E,), jnp.int32),
                       pltpu.VMEM((CAP_E,), jnp.int32),
                       pltpu.VMEM((CAP_E,), jnp.float32),
                       pltpu.VMEM((32,), jnp.int32),
                       pltpu.VMEM((CHUNK,), jnp.int32),
                       pltpu.VMEM((CHUNK,), jnp.int32),
                       pltpu.VMEM((CHUNK,), jnp.int32),
                       pltpu.VMEM((ZERO_ROWS, D_FEAT), jnp.float32),
                       pltpu.SemaphoreType.DMA,
                       pltpu.SemaphoreType.DMA,
                       pltpu.SemaphoreType.DMA,
                       pltpu.SemaphoreType.DMA,
                       pltpu.SemaphoreType.DMA,
                       pltpu.SemaphoreType.DMA,
                       pltpu.SemaphoreType.DMA,
                       pltpu.SemaphoreType.DMA,
                   ])
def _lift(xp, valp, rowp, colp, kb, out, acc, g0, g1, g2, rowv, colv, valv,
          kbv, lrow0, lrow1, lrow2, zerov, semi, semp, semg0, semg1,
          semg2, semsc0, semsc1, semsc2):
    _body(xp, valp, rowp, colp, kb, out, acc, g0, g1, g2, rowv, colv, valv,
          kbv, lrow0, lrow1, lrow2, zerov, semi, semp, semg0, semg1,
          semg2, semsc0, semsc1, semsc2)


def kernel(x_pool, values, row, col):
    row = row.astype(jnp.int32)
    col = col.astype(jnp.int32)
    pad = PAD_ARR - NNZ
    rowp = jnp.concatenate([row, jnp.full((pad,), N_NODES, jnp.int32)])
    colp = jnp.concatenate([col, jnp.zeros((pad,), jnp.int32)])
    valp = jnp.concatenate([values, jnp.zeros((pad,), jnp.float32)])
    cf = rowp[:PADN:CHUNK]
    cl = rowp[CHUNK - 1:PADN:CHUNK]
    sec_lo = jnp.arange(N_SEC, dtype=jnp.int32) * SEC_ROWS
    k_lo = jnp.searchsorted(cl, sec_lo, side="left").astype(jnp.int32)
    k_hi = jnp.searchsorted(cf, sec_lo + SEC_ROWS,
                            side="left").astype(jnp.int32)
    k_hi = jnp.minimum(k_hi, (NNZ + CHUNK - 1) // CHUNK)
    s_of = jnp.array([[0, 2, 4, 6, 8, 10, 12], [1, 3, 5, 7, 9, 11, 13]],
                     dtype=jnp.int32)
    kb = jnp.stack([k_lo[s_of], k_hi[s_of]], axis=-1).reshape(2, 14)
    kb = jnp.pad(kb, ((0, 0), (0, 2))).reshape(32)
    return _lift(x_pool, valp, rowp, colp, kb)
